# Initial kernel scaffold; baseline (speedup 1.0000x reference)
import functools

import jax
import jax.numpy as jnp
from jax import lax
from jax.experimental import pallas as pl
from jax.experimental.pallas import tpu as pltpu
from jax.experimental.pallas import tpu_sc as plsc

N = 10000
E = 320000
D = 128
H = 512

NC = 2
NS = 16
NW = NC * NS

CH = 128
NCH = 80
EPW_P = NCH * CH
TRASH = N
ACC_ROWS = N + 16
ZR = ACC_ROWS // NS
WR = N // NS


def _sc_segment_sum(x, src_r, dst_r, zrows):
  mesh = plsc.VectorSubcoreMesh(
      core_axis_name="c", subcore_axis_name="s", num_cores=NC,
      num_subcores=NS)

  @functools.partial(
      pl.kernel,
      out_type=jax.ShapeDtypeStruct((NC, N, D), jnp.float32),
      mesh=mesh,
      scratch_types=[
          pltpu.VMEM((NCH, CH), jnp.int32),
          pltpu.VMEM((NCH, CH), jnp.int32),
          pltpu.VMEM((CH, D), jnp.float32),
          pltpu.VMEM((CH, D), jnp.float32),
          pltpu.VMEM_SHARED((ACC_ROWS, D), jnp.float32),
          pltpu.SemaphoreType.DMA,
          pltpu.SemaphoreType.DMA,
      ],
  )
  def seg_sum(x_hbm, src_hbm, dst_hbm, zero_hbm, out_hbm,
              src_v, dst_v, g0, g1, acc, sem0, sem1):
    c = lax.axis_index("c")
    s = lax.axis_index("s")
    wid = s * NC + c

    pltpu.sync_copy(src_hbm.at[wid], src_v)
    pltpu.sync_copy(dst_hbm.at[wid], dst_v)
    pltpu.sync_copy(zero_hbm, acc.at[pl.ds(s * ZR, ZR)])
    plsc.subcore_barrier()

    pltpu.async_copy(x_hbm.at[src_v.at[0]], g0, sem0)

    def pair(p, _):
      j0 = 2 * p
      j1 = j0 + 1
      pltpu.async_copy(x_hbm.at[src_v.at[j1]], g1, sem1)
      pltpu.make_async_copy(x_hbm.at[src_v.at[j0]], g0, sem0).wait()
      pltpu.sync_copy(g0, acc.at[dst_v.at[j0]], add=True)

      @pl.when(j1 + 1 < NCH)
      def _fire_next():
        pltpu.async_copy(x_hbm.at[src_v.at[j1 + 1]], g0, sem0)

      pltpu.make_async_copy(x_hbm.at[src_v.at[j1]], g1, sem1).wait()
      pltpu.sync_copy(g1, acc.at[dst_v.at[j1]], add=True)
      return _

    lax.fori_loop(0, NCH // 2, pair, None)
    plsc.subcore_barrier()

    pltpu.sync_copy(acc.at[pl.ds(s * WR, WR)],
                    out_hbm.at[c, pl.ds(s * WR, WR)])

  return seg_sum(x, src_r, dst_r, zrows)


BM = 1000


def _mlp_body(u_ref, x_ref, w1_ref, b1_ref, w2_ref, b2_ref, g_ref, bt_ref,
              o_ref):
  u = u_ref[0] + u_ref[1]
  h1 = jnp.dot(u, w1_ref[...], preferred_element_type=jnp.float32)
  h1 = jnp.maximum(h1 + b1_ref[...], 0.0)
  h = jnp.dot(h1, w2_ref[...], preferred_element_type=jnp.float32)
  y = h + b2_ref[...] + x_ref[...]
  mean = jnp.mean(y, axis=1, keepdims=True)
  yc = y - mean
  var = jnp.mean(yc * yc, axis=1, keepdims=True)
  o_ref[...] = yc * lax.rsqrt(var + 1e-5) * g_ref[...] + bt_ref[...]


def _mlp_ln(u_part, x, W1, b1, W2, b2, gamma, beta):
  grid = (N // BM,)
  return pl.pallas_call(
      _mlp_body,
      grid=grid,
      in_specs=[
          pl.BlockSpec((NC, BM, D), lambda i: (0, i, 0)),
          pl.BlockSpec((BM, D), lambda i: (i, 0)),
          pl.BlockSpec((D, H), lambda i: (0, 0)),
          pl.BlockSpec((1, H), lambda i: (0, 0)),
          pl.BlockSpec((H, D), lambda i: (0, 0)),
          pl.BlockSpec((1, D), lambda i: (0, 0)),
          pl.BlockSpec((1, D), lambda i: (0, 0)),
          pl.BlockSpec((1, D), lambda i: (0, 0)),
      ],
      out_specs=pl.BlockSpec((BM, D), lambda i: (i, 0)),
      out_shape=jax.ShapeDtypeStruct((N, D), jnp.float32),
  )(u_part, x, W1, b1.reshape(1, H), W2, b2.reshape(1, D),
    gamma.reshape(1, D), beta.reshape(1, D))


def kernel(x, edge_index, W1, b1, W2, b2, gamma, beta):
  ei = edge_index.astype(jnp.int32)
  src = ei[0].reshape(NW, E // NW)
  dst = ei[1].reshape(NW, E // NW)
  pad = EPW_P - E // NW
  src_r = jnp.pad(src, ((0, 0), (0, pad))).reshape(NW, NCH, CH)
  dst_r = jnp.pad(dst, ((0, 0), (0, pad)),
                  constant_values=TRASH).reshape(NW, NCH, CH)
  zrows = jnp.zeros((ZR, D), jnp.float32)
  u_part = _sc_segment_sum(x, src_r, dst_r, zrows)
  return _mlp_ln(u_part, x, W1, b1, W2, b2, gamma, beta)


# speedup vs baseline: 5.6123x; 5.6123x over previous
import functools

import jax
import jax.numpy as jnp
from jax import lax
from jax.experimental import pallas as pl
from jax.experimental.pallas import tpu as pltpu
from jax.experimental.pallas import tpu_sc as plsc

N = 10000
E = 320000
D = 128
H = 512

NC = 2
NS = 16
DH = D // NC

CH = 128
EPT = E // NS
NCH = 160
EPT_P = NCH * CH
TRASH = N
ACC_ROWS = 10240
ZR = ACC_ROWS // NS


def _sc_segment_sum(x_cat, src_r, dst_r, zrows):
  mesh = plsc.VectorSubcoreMesh(
      core_axis_name="c", subcore_axis_name="s", num_cores=NC,
      num_subcores=NS)

  @functools.partial(
      pl.kernel,
      out_type=jax.ShapeDtypeStruct((NC, ACC_ROWS, DH), jnp.float32),
      mesh=mesh,
      scratch_types=[
          pltpu.VMEM((NCH, CH), jnp.int32),
          pltpu.VMEM((NCH, CH), jnp.int32),
          pltpu.VMEM((CH, DH), jnp.float32),
          pltpu.VMEM((CH, DH), jnp.float32),
          pltpu.VMEM_SHARED((ACC_ROWS, DH), jnp.float32),
          pltpu.SemaphoreType.DMA,
          pltpu.SemaphoreType.DMA,
      ],
      compiler_params=pltpu.CompilerParams(use_tc_tiling_on_sc=False),
  )
  def seg_sum(x_hbm, src_hbm, dst_hbm, zero_hbm, out_hbm,
              src_v, dst_v, g0, g1, acc, sem0, sem1):
    c = lax.axis_index("c")
    s = lax.axis_index("s")

    pltpu.sync_copy(src_hbm.at[c, s], src_v)
    pltpu.sync_copy(dst_hbm.at[s], dst_v)
    pltpu.sync_copy(zero_hbm, acc.at[pl.ds(s * ZR, ZR)])
    plsc.subcore_barrier()

    pltpu.async_copy(x_hbm.at[src_v.at[0]], g0, sem0)

    def pair(p, _):
      j0 = 2 * p
      j1 = j0 + 1
      pltpu.async_copy(x_hbm.at[src_v.at[j1]], g1, sem1)
      pltpu.make_async_copy(x_hbm.at[src_v.at[j0]], g0, sem0).wait()
      pltpu.sync_copy(g0, acc.at[dst_v.at[j0]], add=True)

      @pl.when(j1 + 1 < NCH)
      def _fire_next():
        pltpu.async_copy(x_hbm.at[src_v.at[j1 + 1]], g0, sem0)

      pltpu.make_async_copy(x_hbm.at[src_v.at[j1]], g1, sem1).wait()
      pltpu.sync_copy(g1, acc.at[dst_v.at[j1]], add=True)
      return _

    lax.fori_loop(0, NCH // 2, pair, None)
    plsc.subcore_barrier()

    pltpu.sync_copy(acc.at[pl.ds(s * ZR, ZR)],
                    out_hbm.at[c, pl.ds(s * ZR, ZR)])

  return seg_sum(x_cat, src_r, dst_r, zrows)


BM = 1000


def _mlp_body(u_ref, x_ref, w1_ref, b1_ref, w2_ref, b2_ref, g_ref, bt_ref,
              o_ref):
  u = jnp.concatenate([u_ref[0], u_ref[1]], axis=1)
  h1 = jnp.dot(u, w1_ref[...], preferred_element_type=jnp.float32)
  h1 = jnp.maximum(h1 + b1_ref[...], 0.0)
  h = jnp.dot(h1, w2_ref[...], preferred_element_type=jnp.float32)
  y = h + b2_ref[...] + x_ref[...]
  mean = jnp.mean(y, axis=1, keepdims=True)
  yc = y - mean
  var = jnp.mean(yc * yc, axis=1, keepdims=True)
  o_ref[...] = yc * lax.rsqrt(var + 1e-5) * g_ref[...] + bt_ref[...]


def _mlp_ln(u_part, x, W1, b1, W2, b2, gamma, beta):
  grid = (N // BM,)
  return pl.pallas_call(
      _mlp_body,
      grid=grid,
      in_specs=[
          pl.BlockSpec((NC, BM, DH), lambda i: (0, i, 0)),
          pl.BlockSpec((BM, D), lambda i: (i, 0)),
          pl.BlockSpec((D, H), lambda i: (0, 0)),
          pl.BlockSpec((1, H), lambda i: (0, 0)),
          pl.BlockSpec((H, D), lambda i: (0, 0)),
          pl.BlockSpec((1, D), lambda i: (0, 0)),
          pl.BlockSpec((1, D), lambda i: (0, 0)),
          pl.BlockSpec((1, D), lambda i: (0, 0)),
      ],
      out_specs=pl.BlockSpec((BM, D), lambda i: (i, 0)),
      out_shape=jax.ShapeDtypeStruct((N, D), jnp.float32),
  )(u_part, x, W1, b1.reshape(1, H), W2, b2.reshape(1, D),
    gamma.reshape(1, D), beta.reshape(1, D))


def kernel(x, edge_index, W1, b1, W2, b2, gamma, beta):
  ei = edge_index.astype(jnp.int32)
  pad = EPT_P - EPT
  src = jnp.pad(ei[0].reshape(NS, EPT), ((0, 0), (0, pad)))
  dst = jnp.pad(ei[1].reshape(NS, EPT), ((0, 0), (0, pad)),
                constant_values=TRASH)
  src_r = jnp.stack([src, src + N]).reshape(NC, NS, NCH, CH)
  dst_r = dst.reshape(NS, NCH, CH)
  x_cat = jnp.concatenate([x[:, :DH], x[:, DH:]], axis=0)
  zrows = jnp.zeros((ZR, DH), jnp.float32)
  u_part = _sc_segment_sum(x_cat, src_r, dst_r, zrows)
  return _mlp_ln(u_part, x, W1, b1, W2, b2, gamma, beta)


# speedup vs baseline: 6.0106x; 1.0710x over previous
import functools

import jax
import jax.numpy as jnp
from jax import lax
from jax.experimental import pallas as pl
from jax.experimental.pallas import tpu as pltpu
from jax.experimental.pallas import tpu_sc as plsc

N = 10000
E = 320000
D = 128
H = 512

NC = 2
NS = 16
DH = D // NC

CH = 128
EPT = E // NS
NCH = 160
EPT_P = NCH * CH
TRASH = N
ACC_ROWS = 10240
ZR = ACC_ROWS // NS


def _sc_segment_sum(x_cat, src_r, dst_r, zrows):
  mesh = plsc.VectorSubcoreMesh(
      core_axis_name="c", subcore_axis_name="s", num_cores=NC,
      num_subcores=NS)

  @functools.partial(
      pl.kernel,
      out_type=jax.ShapeDtypeStruct((NC, ACC_ROWS, DH), jnp.float32),
      mesh=mesh,
      scratch_types=[
          pltpu.VMEM((NCH, CH), jnp.int32),
          pltpu.VMEM((NCH, CH), jnp.int32),
          pltpu.VMEM((CH, DH), jnp.float32),
          pltpu.VMEM((CH, DH), jnp.float32),
          pltpu.VMEM((CH, DH), jnp.float32),
          pltpu.VMEM((CH, DH), jnp.float32),
          pltpu.SemaphoreType.DMA,
          pltpu.SemaphoreType.DMA,
          pltpu.SemaphoreType.DMA,
          pltpu.SemaphoreType.DMA,
          pltpu.SemaphoreType.DMA,
          pltpu.SemaphoreType.DMA,
          pltpu.SemaphoreType.DMA,
          pltpu.SemaphoreType.DMA,
          pltpu.VMEM_SHARED((ACC_ROWS, DH), jnp.float32),
      ],
      compiler_params=pltpu.CompilerParams(use_tc_tiling_on_sc=False),
  )
  def seg_sum(x_hbm, src_hbm, dst_hbm, zero_hbm, out_hbm,
              src_v, dst_v, g0, g1, g2, g3,
              gs0, gs1, gs2, gs3, ss0, ss1, ss2, ss3, acc):
    c = lax.axis_index("c")
    s = lax.axis_index("s")
    gb = [g0, g1, g2, g3]
    gsem = [gs0, gs1, gs2, gs3]
    ssem = [ss0, ss1, ss2, ss3]

    pltpu.sync_copy(src_hbm.at[c, s], src_v)
    pltpu.sync_copy(dst_hbm.at[s], dst_v)
    pltpu.sync_copy(zero_hbm, acc.at[pl.ds(s * ZR, ZR)])
    plsc.subcore_barrier()

    for t in range(3):
      pltpu.async_copy(x_hbm.at[src_v.at[t]], gb[t], gsem[t])

    def quad(p, _):
      for t in range(4):
        j = 4 * p + t
        pltpu.make_async_copy(x_hbm.at[src_v.at[j]], gb[t], gsem[t]).wait()
        pltpu.async_copy(gb[t], acc.at[dst_v.at[j]], ssem[t], add=True)
        tn = (t + 3) % 4

        @pl.when(j >= 1)
        def _drain_prev():
          pltpu.make_async_copy(gb[tn], acc.at[dst_v.at[j]],
                                ssem[tn]).wait()

        @pl.when(j + 3 < NCH)
        def _fire_next():
          pltpu.async_copy(x_hbm.at[src_v.at[j + 3]], gb[tn], gsem[tn])

      return _

    lax.fori_loop(0, NCH // 4, quad, None)
    pltpu.make_async_copy(gb[3], acc.at[dst_v.at[NCH - 1]], ssem[3]).wait()
    plsc.subcore_barrier()

    pltpu.sync_copy(acc.at[pl.ds(s * ZR, ZR)],
                    out_hbm.at[c, pl.ds(s * ZR, ZR)])

  return seg_sum(x_cat, src_r, dst_r, zrows)


BM = 1000


def _mlp_body(u_ref, x_ref, w1_ref, b1_ref, w2_ref, b2_ref, g_ref, bt_ref,
              o_ref):
  u = jnp.concatenate([u_ref[0], u_ref[1]], axis=1)
  h1 = jnp.dot(u, w1_ref[...], preferred_element_type=jnp.float32)
  h1 = jnp.maximum(h1 + b1_ref[...], 0.0)
  h = jnp.dot(h1, w2_ref[...], preferred_element_type=jnp.float32)
  y = h + b2_ref[...] + x_ref[...]
  mean = jnp.mean(y, axis=1, keepdims=True)
  yc = y - mean
  var = jnp.mean(yc * yc, axis=1, keepdims=True)
  o_ref[...] = yc * lax.rsqrt(var + 1e-5) * g_ref[...] + bt_ref[...]


def _mlp_ln(u_part, x, W1, b1, W2, b2, gamma, beta):
  grid = (N // BM,)
  return pl.pallas_call(
      _mlp_body,
      grid=grid,
      in_specs=[
          pl.BlockSpec((NC, BM, DH), lambda i: (0, i, 0)),
          pl.BlockSpec((BM, D), lambda i: (i, 0)),
          pl.BlockSpec((D, H), lambda i: (0, 0)),
          pl.BlockSpec((1, H), lambda i: (0, 0)),
          pl.BlockSpec((H, D), lambda i: (0, 0)),
          pl.BlockSpec((1, D), lambda i: (0, 0)),
          pl.BlockSpec((1, D), lambda i: (0, 0)),
          pl.BlockSpec((1, D), lambda i: (0, 0)),
      ],
      out_specs=pl.BlockSpec((BM, D), lambda i: (i, 0)),
      out_shape=jax.ShapeDtypeStruct((N, D), jnp.float32),
  )(u_part, x, W1, b1.reshape(1, H), W2, b2.reshape(1, D),
    gamma.reshape(1, D), beta.reshape(1, D))


def kernel(x, edge_index, W1, b1, W2, b2, gamma, beta):
  ei = edge_index.astype(jnp.int32)
  pad = EPT_P - EPT
  src = jnp.pad(ei[0].reshape(NS, EPT), ((0, 0), (0, pad)))
  dst = jnp.pad(ei[1].reshape(NS, EPT), ((0, 0), (0, pad)),
                constant_values=TRASH)
  src_r = jnp.stack([src, src + N]).reshape(NC, NS, NCH, CH)
  dst_r = dst.reshape(NS, NCH, CH)
  x_cat = jnp.concatenate([x[:, :DH], x[:, DH:]], axis=0)
  zrows = jnp.zeros((ZR, DH), jnp.float32)
  u_part = _sc_segment_sum(x_cat, src_r, dst_r, zrows)
  return _mlp_ln(u_part, x, W1, b1, W2, b2, gamma, beta)
